# baseline (speedup 1.0000x reference)
import jax
import jax.numpy as jnp
from jax import lax
from jax.experimental import pallas as pl
from jax.experimental.pallas import tpu as pltpu

_E = 8
_TOPK = 2
_T = 64
_D = 1024
_F = 4096
_BF = 1024
_NF = _F // _BF
_CH = 256


def _moe_body(rt_ref, rw_ref, x_ref, wg_ref, wu_ref, w2_ref, out_ref, tw_ref):
    i = pl.program_id(0)
    e = i // _NF

    @pl.when(i == 0)
    def _init():
        out_ref[...] = jnp.zeros_like(out_ref)
        ei = lax.broadcasted_iota(jnp.int32, (_E, _T, _TOPK), 0)
        rt3 = rt_ref[...][None]
        rw3 = rw_ref[...][None]
        tw_ref[...] = jnp.sum(jnp.where(rt3 == ei, rw3, 0.0), axis=2)

    tok_w = tw_ref[e, :][:, None]

    x = x_ref[...].astype(jnp.bfloat16)

    acc = None
    for c in range(0, _BF, _CH):
        wg = wg_ref[0, c:c + _CH, :].astype(jnp.bfloat16)
        wu = wu_ref[0, c:c + _CH, :].astype(jnp.bfloat16)
        w2b = w2_ref[0, :, c:c + _CH].astype(jnp.bfloat16)
        g = jax.lax.dot_general(x, wg, (((1,), (1,)), ((), ())),
                                preferred_element_type=jnp.float32)
        u = jax.lax.dot_general(x, wu, (((1,), (1,)), ((), ())),
                                preferred_element_type=jnp.float32)
        h = (g * jax.nn.sigmoid(g)) * u * tok_w
        o = jax.lax.dot_general(h.astype(jnp.bfloat16), w2b,
                                (((1,), (1,)), ((), ())),
                                preferred_element_type=jnp.float32)
        acc = o if acc is None else acc + o
    out_ref[...] += acc


def kernel(hidden_states, expert_routing_table, router_weights, w13, w2):
    rt = expert_routing_table.astype(jnp.int32)
    return pl.pallas_call(
        _moe_body,
        grid=(_E * _NF,),
        in_specs=[
            pl.BlockSpec((_T, _TOPK), lambda i: (0, 0)),
            pl.BlockSpec((_T, _TOPK), lambda i: (0, 0)),
            pl.BlockSpec((_T, _D), lambda i: (0, 0)),
            pl.BlockSpec((1, _BF, _D), lambda i: (i // _NF, i % _NF, 0)),
            pl.BlockSpec((1, _BF, _D), lambda i: (i // _NF, _NF + i % _NF, 0)),
            pl.BlockSpec((1, _D, _BF), lambda i: (i // _NF, 0, i % _NF)),
        ],
        out_specs=pl.BlockSpec((_T, _D), lambda i: (0, 0)),
        out_shape=jax.ShapeDtypeStruct((_T, _D), jnp.float32),
        scratch_shapes=[pltpu.VMEM((_E, _T), jnp.float32)],
    )(rt, router_weights, hidden_states, w13, w13, w2)


# speedup vs baseline: 1.0029x; 1.0029x over previous
import jax
import jax.numpy as jnp
from jax import lax
from jax.experimental import pallas as pl
from jax.experimental.pallas import tpu as pltpu

_E = 8
_TOPK = 2
_T = 64
_D = 1024
_F = 4096
_BF = 1024
_NF = _F // _BF
_CH = 256


def _moe_body(rt_ref, rw_ref, x_ref, wg_ref, wu_ref, w2_ref, out_ref,
              tw_ref, xb_ref):
    i = pl.program_id(0)
    e = i // _NF

    @pl.when(i == 0)
    def _init():
        out_ref[...] = jnp.zeros_like(out_ref)
        xb_ref[...] = x_ref[...].astype(jnp.bfloat16)
        ei = lax.broadcasted_iota(jnp.int32, (_E, _T, _TOPK), 0)
        rt3 = rt_ref[...][None]
        rw3 = rw_ref[...][None]
        tw_ref[...] = jnp.sum(jnp.where(rt3 == ei, rw3, 0.0), axis=2)

    tok_w = tw_ref[e, :][:, None]

    x = xb_ref[...]
    wg = wg_ref[0].astype(jnp.bfloat16)
    wu = wu_ref[0].astype(jnp.bfloat16)
    w2b = w2_ref[0].astype(jnp.bfloat16)

    g = jax.lax.dot_general(x, wg, (((1,), (1,)), ((), ())),
                            preferred_element_type=jnp.float32)
    u = jax.lax.dot_general(x, wu, (((1,), (1,)), ((), ())),
                            preferred_element_type=jnp.float32)
    h = (g * jax.nn.sigmoid(g)) * u * tok_w
    o = jax.lax.dot_general(h.astype(jnp.bfloat16), w2b,
                            (((1,), (1,)), ((), ())),
                            preferred_element_type=jnp.float32)
    out_ref[...] += o


def kernel(hidden_states, expert_routing_table, router_weights, w13, w2):
    rt = expert_routing_table.astype(jnp.int32)
    return pl.pallas_call(
        _moe_body,
        grid=(_E * _NF,),
        in_specs=[
            pl.BlockSpec((_T, _TOPK), lambda i: (0, 0)),
            pl.BlockSpec((_T, _TOPK), lambda i: (0, 0)),
            pl.BlockSpec((_T, _D), lambda i: (0, 0)),
            pl.BlockSpec((1, _BF, _D), lambda i: (i // _NF, i % _NF, 0)),
            pl.BlockSpec((1, _BF, _D), lambda i: (i // _NF, _NF + i % _NF, 0)),
            pl.BlockSpec((1, _D, _BF), lambda i: (i // _NF, 0, i % _NF)),
        ],
        out_specs=pl.BlockSpec((_T, _D), lambda i: (0, 0)),
        out_shape=jax.ShapeDtypeStruct((_T, _D), jnp.float32),
        scratch_shapes=[pltpu.VMEM((_E, _T), jnp.float32),
                        pltpu.VMEM((_T, _D), jnp.bfloat16)],
    )(rt, router_weights, hidden_states, w13, w13, w2)
